# baseline (speedup 1.0000x reference)
import functools

import jax
import jax.numpy as jnp
import numpy as np
from jax import lax
from jax.experimental import pallas as pl
from jax.experimental.pallas import tpu as pltpu
from jax.experimental.pallas import tpu_sc as plsc

N_NODES = 10000
N_EDGES = 320000
D_FEAT = 128
LAYERS = 3
ALPHA = 0.5

NC = 2
NS = 16
DH = D_FEAT // NC

CE = 96
CPT = 211
EPT = CE * CPT
E_PAD = EPT * NS

N_PAD = 10240
RPT = N_PAD // NS
RCH = 128
ZR = 64

_PERM = np.empty((DH,), np.int32)
for _g in range(DH // 32):
    for _k in range(16):
        _PERM[_g * 32 + 2 * _k] = _g * 32 + _k
        _PERM[_g * 32 + 2 * _k + 1] = _g * 32 + 16 + _k

_sc_mesh = plsc.VectorSubcoreMesh(
    core_axis_name="c", subcore_axis_name="s", num_cores=NC, num_subcores=NS)


@functools.partial(
    pl.kernel,
    out_type=jax.ShapeDtypeStruct((NC, N_PAD, DH), jnp.float32),
    mesh=_sc_mesh,
    scratch_types=[
        pltpu.VMEM((CPT, CE), jnp.int32),
        pltpu.VMEM((CPT, CE), jnp.int32),
        pltpu.VMEM((CPT, CE), jnp.float32),
        pltpu.VMEM((CE, DH), jnp.bfloat16),
        pltpu.VMEM((CE, DH), jnp.bfloat16),
        pltpu.VMEM((CE, DH), jnp.bfloat16),
        pltpu.VMEM((CE, DH), jnp.float32),
        pltpu.VMEM((CE, DH), jnp.float32),
        pltpu.VMEM((CE, DH), jnp.float32),
        pltpu.SemaphoreType.DMA,
        pltpu.SemaphoreType.DMA,
        pltpu.SemaphoreType.DMA,
        pltpu.SemaphoreType.DMA,
        pltpu.SemaphoreType.DMA,
        pltpu.SemaphoreType.DMA,
        pltpu.VMEM_SHARED((N_PAD, DH), jnp.float32),
    ],
    compiler_params=pltpu.CompilerParams(
        use_tc_tiling_on_sc=False, needs_layout_passes=False),
)
def _sc_agg(xs_hbm, src_hbm, dst_hbm, w_hbm, out_hbm,
            src_v, dst_v, w_v, gb0, gb1, gb2, sb0, sb1, sb2,
            g0, g1, g2, u0, u1, u2, acc_sh):
    c = lax.axis_index("c")
    s = lax.axis_index("s")
    gbufs = (gb0, gb1, gb2)
    sbufs = (sb0, sb1, sb2)
    gsems = (g0, g1, g2)
    usems = (u0, u1, u2)
    table = xs_hbm.at[c]

    def _zero_row(r, _):
        for f in range(DH // 16):
            sb0[r, pl.ds(f * 16, 16)] = jnp.zeros((16,), jnp.float32)
        return 0
    lax.fori_loop(0, ZR, _zero_row, 0)
    zsrc = sb0.at[pl.ds(0, ZR)]
    for k in range(RPT // ZR):
        pltpu.sync_copy(zsrc, acc_sh.at[pl.ds(s * RPT + k * ZR, ZR)])
    plsc.subcore_barrier()

    pltpu.sync_copy(src_hbm.at[s], src_v)
    pltpu.sync_copy(dst_hbm.at[s], dst_v)
    pltpu.sync_copy(w_hbm.at[s], w_v)

    def _issue_gather(t, j):
        pltpu.async_copy(table.at[src_v.at[t]], gbufs[j], gsems[j])

    def _wait_gather(t, j):
        pltpu.make_async_copy(table.at[src_v.at[t]], gbufs[j], gsems[j]).wait()

    def _issue_scatter(t, j):
        pltpu.async_copy(sbufs[j], acc_sh.at[dst_v.at[t]], usems[j], add=True)

    def _wait_scatter(t, j):
        pltpu.make_async_copy(
            sbufs[j], acc_sh.at[dst_v.at[t]], usems[j]).wait()

    def _scale(t, j):
        gbuf = gbufs[j]
        sbuf = sbufs[j]

        @plsc.parallel_loop(0, CE, 2, unroll=2)
        def _scale2(e0):
            for jj in range(2):
                e = e0 + jj
                w_splat = plsc.load_gather(
                    w_v.at[t], [jnp.full((16,), e, jnp.int32)])
                for g in range(DH // 32):
                    raw = gbuf[e, pl.ds(g * 32, 32)]
                    a, b = plsc.unpack(
                        raw, format=plsc.PackFormat.INTERLEAVED,
                        preferred_element_type=jnp.float32)
                    sbuf[e, pl.ds(g * 32, 16)] = a * w_splat
                    sbuf[e, pl.ds(g * 32 + 16, 16)] = b * w_splat

    _issue_gather(0, 0)
    _issue_gather(1, 1)
    _issue_gather(2, 2)

    for t0 in range(3):
        _wait_gather(t0, t0)
        _scale(t0, t0)
        _issue_scatter(t0, t0)
        _issue_gather(t0 + 3, t0)

    def _body(v, _):
        t = v * 3
        for j in range(3):
            _wait_gather(t + j, j)
            _wait_scatter(t + j - 3, j)
            _scale(t + j, j)
            _issue_scatter(t + j, j)

            @pl.when(t + j + 3 < CPT)
            def _():
                _issue_gather(t + j + 3, j)
        return 0
    lax.fori_loop(1, CPT // 3, _body, 0)

    _wait_gather(CPT - 1, 0)
    _wait_scatter(CPT - 4, 0)
    _scale(CPT - 1, 0)
    _issue_scatter(CPT - 1, 0)
    _wait_scatter(CPT - 3, 1)
    _wait_scatter(CPT - 2, 2)
    _wait_scatter(CPT - 1, 0)

    plsc.subcore_barrier()
    for k in range(RPT // RCH):
        base = s * RPT + k * RCH
        pltpu.sync_copy(acc_sh.at[pl.ds(base, RCH)],
                        out_hbm.at[c].at[pl.ds(base, RCH)])


_RB = 1000


def _perm_interleave(hw):
    parts = []
    for g in range(DH // 32):
        a = hw[:, g * 32:g * 32 + 16]
        b = hw[:, g * 32 + 16:g * 32 + 32]
        parts.append(jnp.stack([a, b], axis=-1).reshape(hw.shape[0], 32))
    return jnp.concatenate(parts, axis=1)


def _split_perm_bf16(h0, h1, w_ref, xsn_ref):
    xsn_ref[0] = _perm_interleave(h0 * w_ref[:, :DH]).astype(jnp.bfloat16)
    xsn_ref[1] = _perm_interleave(h1 * w_ref[:, DH:]).astype(jnp.bfloat16)


def _prescale_body(x_ref, w_ref, xs_ref):
    _split_perm_bf16(x_ref[:, :DH], x_ref[:, DH:], w_ref, xs_ref)


def _elem_body(x_ref, agg_ref, deg_ref, w_ref, xn_ref, xsn_ref):
    d = deg_ref[...]
    scale = (1.0 - ALPHA) * d * d
    h0 = ALPHA * x_ref[:, :DH] + scale * agg_ref[0]
    h1 = ALPHA * x_ref[:, DH:] + scale * agg_ref[1]
    inv = 1.0 / (jnp.sum(h0, axis=1, keepdims=True)
                 + jnp.sum(h1, axis=1, keepdims=True))
    h0 = h0 * inv
    h1 = h1 * inv
    xn_ref[:, :DH] = h0
    xn_ref[:, DH:] = h1
    _split_perm_bf16(h0, h1, w_ref, xsn_ref)


def _prescale(x, W):
    return pl.pallas_call(
        _prescale_body,
        grid=(N_NODES // _RB,),
        in_specs=[
            pl.BlockSpec((_RB, D_FEAT), lambda i: (i, 0)),
            pl.BlockSpec((1, D_FEAT), lambda i: (0, 0)),
        ],
        out_specs=pl.BlockSpec((NC, _RB, DH), lambda i: (0, i, 0)),
        out_shape=jax.ShapeDtypeStruct((NC, N_NODES, DH), jnp.bfloat16),
    )(x, W)


def _elemwise(x, agg, deg_col, W):
    return pl.pallas_call(
        _elem_body,
        grid=(N_NODES // _RB,),
        in_specs=[
            pl.BlockSpec((_RB, D_FEAT), lambda i: (i, 0)),
            pl.BlockSpec((NC, _RB, DH), lambda i: (0, i, 0)),
            pl.BlockSpec((_RB, 1), lambda i: (i, 0)),
            pl.BlockSpec((1, D_FEAT), lambda i: (0, 0)),
        ],
        out_specs=[
            pl.BlockSpec((_RB, D_FEAT), lambda i: (i, 0)),
            pl.BlockSpec((NC, _RB, DH), lambda i: (0, i, 0)),
        ],
        out_shape=[
            jax.ShapeDtypeStruct((N_NODES, D_FEAT), jnp.float32),
            jax.ShapeDtypeStruct((NC, N_NODES, DH), jnp.bfloat16),
        ],
    )(x, agg, deg_col, W)


@jax.jit
def kernel(x, edge_index, edge_weight, degree, W):
    src = edge_index[0].astype(jnp.int32)
    dst = edge_index[1].astype(jnp.int32)
    pad = E_PAD - N_EDGES
    src3 = jnp.pad(src, (0, pad)).reshape(NS, CPT, CE)
    dst3 = jnp.pad(dst, (0, pad)).reshape(NS, CPT, CE)
    w3 = jnp.pad(edge_weight, (0, pad)).reshape(NS, CPT, CE)
    deg_col = degree[:, None]

    xs = _prescale(x, W)
    for _ in range(LAYERS):
        agg = _sc_agg(xs, src3, dst3, w3)
        x, xs = _elemwise(x, agg, deg_col, W)
    return x


# speedup vs baseline: 2.4671x; 2.4671x over previous
import functools

import jax
import jax.numpy as jnp
from jax import lax
from jax.experimental import pallas as pl
from jax.experimental.pallas import tpu as pltpu
from jax.experimental.pallas import tpu_sc as plsc

N_NODES = 10000
N_EDGES = 320000
D_FEAT = 128
LAYERS = 3
ALPHA = 0.5

NC = 2
NS = 16
DH = D_FEAT // NC

CE = 96
CPT = 211
EPT = CE * CPT
E_PAD = EPT * NS

N_PAD = 10240
RPT = N_PAD // NS
RCH = 128
ZR = 64

_sc_mesh = plsc.VectorSubcoreMesh(
    core_axis_name="c", subcore_axis_name="s", num_cores=NC, num_subcores=NS)


@functools.partial(
    pl.kernel,
    out_type=jax.ShapeDtypeStruct((NC, N_PAD, DH), jnp.float32),
    mesh=_sc_mesh,
    scratch_types=[
        pltpu.VMEM((CPT, CE), jnp.int32),
        pltpu.VMEM((CPT, CE), jnp.int32),
        pltpu.VMEM((CPT, CE), jnp.float32),
        pltpu.VMEM((CE, DH), jnp.bfloat16),
        pltpu.VMEM((CE, DH), jnp.bfloat16),
        pltpu.VMEM((CE, DH), jnp.bfloat16),
        pltpu.VMEM((CE, DH), jnp.float32),
        pltpu.VMEM((CE, DH), jnp.float32),
        pltpu.VMEM((CE, DH), jnp.float32),
        pltpu.SemaphoreType.DMA,
        pltpu.SemaphoreType.DMA,
        pltpu.SemaphoreType.DMA,
        pltpu.SemaphoreType.DMA,
        pltpu.SemaphoreType.DMA,
        pltpu.SemaphoreType.DMA,
        pltpu.VMEM_SHARED((N_PAD, DH), jnp.float32),
    ],
    compiler_params=pltpu.CompilerParams(
        use_tc_tiling_on_sc=False, needs_layout_passes=False),
)
def _sc_agg(xs_hbm, src_hbm, dst_hbm, w_hbm, out_hbm,
            src_v, dst_v, w_v, gb0, gb1, gb2, sb0, sb1, sb2,
            g0, g1, g2, u0, u1, u2, acc_sh):
    c = lax.axis_index("c")
    s = lax.axis_index("s")
    gbufs = (gb0, gb1, gb2)
    sbufs = (sb0, sb1, sb2)
    gsems = (g0, g1, g2)
    usems = (u0, u1, u2)
    table = xs_hbm.at[c]

    def _zero_row(r, _):
        for f in range(DH // 16):
            sb0[r, pl.ds(f * 16, 16)] = jnp.zeros((16,), jnp.float32)
        return 0
    lax.fori_loop(0, ZR, _zero_row, 0)
    zsrc = sb0.at[pl.ds(0, ZR)]
    for k in range(RPT // ZR):
        pltpu.sync_copy(zsrc, acc_sh.at[pl.ds(s * RPT + k * ZR, ZR)])
    plsc.subcore_barrier()

    pltpu.sync_copy(src_hbm.at[s], src_v)
    pltpu.sync_copy(dst_hbm.at[s], dst_v)
    pltpu.sync_copy(w_hbm.at[s], w_v)

    def _issue_gather(t, j):
        pltpu.async_copy(table.at[src_v.at[t]], gbufs[j], gsems[j])

    def _wait_gather(t, j):
        pltpu.make_async_copy(table.at[src_v.at[t]], gbufs[j], gsems[j]).wait()

    def _issue_scatter(t, j):
        pltpu.async_copy(sbufs[j], acc_sh.at[dst_v.at[t]], usems[j], add=True)

    def _wait_scatter(t, j):
        pltpu.make_async_copy(
            sbufs[j], acc_sh.at[dst_v.at[t]], usems[j]).wait()

    ev = 2 * lax.iota(jnp.int32, 16)
    cols = [(ev + g * 32, ev + g * 32 + 1) for g in range(DH // 32)]

    def _scale(t, j):
        gbuf = gbufs[j]
        sbuf = sbufs[j]

        @plsc.parallel_loop(0, CE, 2, unroll=2)
        def _scale2(e0):
            for jj in range(2):
                e = e0 + jj
                splat_e = jnp.full((16,), e, jnp.int32)
                w_splat = plsc.load_gather(w_v.at[t], [splat_e])
                for g in range(DH // 32):
                    raw = gbuf[e, pl.ds(g * 32, 32)]
                    a, b = plsc.unpack(
                        raw, format=plsc.PackFormat.INTERLEAVED,
                        preferred_element_type=jnp.float32)
                    plsc.store_scatter(sbuf, [splat_e, cols[g][0]],
                                       a * w_splat)
                    plsc.store_scatter(sbuf, [splat_e, cols[g][1]],
                                       b * w_splat)

    _issue_gather(0, 0)
    _issue_gather(1, 1)
    _issue_gather(2, 2)

    for t0 in range(3):
        _wait_gather(t0, t0)
        _scale(t0, t0)
        _issue_scatter(t0, t0)
        _issue_gather(t0 + 3, t0)

    def _body(v, _):
        t = v * 3
        for j in range(3):
            _wait_gather(t + j, j)
            _wait_scatter(t + j - 3, j)
            _scale(t + j, j)
            _issue_scatter(t + j, j)

            @pl.when(t + j + 3 < CPT)
            def _():
                _issue_gather(t + j + 3, j)
        return 0
    lax.fori_loop(1, CPT // 3, _body, 0)

    _wait_gather(CPT - 1, 0)
    _wait_scatter(CPT - 4, 0)
    _scale(CPT - 1, 0)
    _issue_scatter(CPT - 1, 0)
    _wait_scatter(CPT - 3, 1)
    _wait_scatter(CPT - 2, 2)
    _wait_scatter(CPT - 1, 0)

    plsc.subcore_barrier()
    for k in range(RPT // RCH):
        base = s * RPT + k * RCH
        pltpu.sync_copy(acc_sh.at[pl.ds(base, RCH)],
                        out_hbm.at[c].at[pl.ds(base, RCH)])


_RB = 1000


def _split_perm_bf16(h0, h1, w_ref, xsn_ref):
    xsn_ref[0] = (h0 * w_ref[:, :DH]).astype(jnp.bfloat16)
    xsn_ref[1] = (h1 * w_ref[:, DH:]).astype(jnp.bfloat16)


def _prescale_body(x_ref, w_ref, xs_ref):
    _split_perm_bf16(x_ref[:, :DH], x_ref[:, DH:], w_ref, xs_ref)


def _elem_body(x_ref, agg_ref, deg_ref, w_ref, xn_ref, xsn_ref):
    d = deg_ref[...]
    scale = (1.0 - ALPHA) * d * d
    h0 = ALPHA * x_ref[:, :DH] + scale * agg_ref[0]
    h1 = ALPHA * x_ref[:, DH:] + scale * agg_ref[1]
    inv = 1.0 / (jnp.sum(h0, axis=1, keepdims=True)
                 + jnp.sum(h1, axis=1, keepdims=True))
    h0 = h0 * inv
    h1 = h1 * inv
    xn_ref[:, :DH] = h0
    xn_ref[:, DH:] = h1
    _split_perm_bf16(h0, h1, w_ref, xsn_ref)


def _prescale(x, W):
    return pl.pallas_call(
        _prescale_body,
        grid=(N_NODES // _RB,),
        in_specs=[
            pl.BlockSpec((_RB, D_FEAT), lambda i: (i, 0)),
            pl.BlockSpec((1, D_FEAT), lambda i: (0, 0)),
        ],
        out_specs=pl.BlockSpec((NC, _RB, DH), lambda i: (0, i, 0)),
        out_shape=jax.ShapeDtypeStruct((NC, N_NODES, DH), jnp.bfloat16),
    )(x, W)


def _elemwise(x, agg, deg_col, W):
    return pl.pallas_call(
        _elem_body,
        grid=(N_NODES // _RB,),
        in_specs=[
            pl.BlockSpec((_RB, D_FEAT), lambda i: (i, 0)),
            pl.BlockSpec((NC, _RB, DH), lambda i: (0, i, 0)),
            pl.BlockSpec((_RB, 1), lambda i: (i, 0)),
            pl.BlockSpec((1, D_FEAT), lambda i: (0, 0)),
        ],
        out_specs=[
            pl.BlockSpec((_RB, D_FEAT), lambda i: (i, 0)),
            pl.BlockSpec((NC, _RB, DH), lambda i: (0, i, 0)),
        ],
        out_shape=[
            jax.ShapeDtypeStruct((N_NODES, D_FEAT), jnp.float32),
            jax.ShapeDtypeStruct((NC, N_NODES, DH), jnp.bfloat16),
        ],
    )(x, agg, deg_col, W)


@jax.jit
def kernel(x, edge_index, edge_weight, degree, W):
    src = edge_index[0].astype(jnp.int32)
    dst = edge_index[1].astype(jnp.int32)
    pad = E_PAD - N_EDGES
    src3 = jnp.pad(src, (0, pad)).reshape(NS, CPT, CE)
    dst3 = jnp.pad(dst, (0, pad)).reshape(NS, CPT, CE)
    w3 = jnp.pad(edge_weight, (0, pad)).reshape(NS, CPT, CE)
    deg_col = degree[:, None]

    xs = _prescale(x, W)
    for _ in range(LAYERS):
        agg = _sc_agg(xs, src3, dst3, w3)
        x, xs = _elemwise(x, agg, deg_col, W)
    return x


# speedup vs baseline: 2.5411x; 1.0300x over previous
import functools

import jax
import jax.numpy as jnp
from jax import lax
from jax.experimental import pallas as pl
from jax.experimental.pallas import tpu as pltpu
from jax.experimental.pallas import tpu_sc as plsc

N_NODES = 10000
N_EDGES = 320000
D_FEAT = 128
LAYERS = 3
ALPHA = 0.5

NC = 2
NS = 16
DH = D_FEAT // NC

CE = 128
CPT = 158
EPT = CE * CPT
E_PAD = EPT * NS
NGB = 3
NSB = 2

N_PAD = 10240
RPT = N_PAD // NS
RCH = 128
ZR = 64

_sc_mesh = plsc.VectorSubcoreMesh(
    core_axis_name="c", subcore_axis_name="s", num_cores=NC, num_subcores=NS)


@functools.partial(
    pl.kernel,
    out_type=jax.ShapeDtypeStruct((NC, N_PAD, DH), jnp.float32),
    mesh=_sc_mesh,
    scratch_types=[
        pltpu.VMEM((CPT, CE), jnp.int32),
        pltpu.VMEM((CPT, CE), jnp.int32),
        pltpu.VMEM((CPT, CE), jnp.float32),
        pltpu.VMEM((CE, DH), jnp.bfloat16),
        pltpu.VMEM((CE, DH), jnp.bfloat16),
        pltpu.VMEM((CE, DH), jnp.bfloat16),
        pltpu.VMEM((CE, DH), jnp.float32),
        pltpu.VMEM((CE, DH), jnp.float32),
        pltpu.SemaphoreType.DMA,
        pltpu.SemaphoreType.DMA,
        pltpu.SemaphoreType.DMA,
        pltpu.SemaphoreType.DMA,
        pltpu.SemaphoreType.DMA,
        pltpu.VMEM_SHARED((N_PAD, DH), jnp.float32),
    ],
    compiler_params=pltpu.CompilerParams(
        use_tc_tiling_on_sc=False, needs_layout_passes=False),
)
def _sc_agg(xs_hbm, src_hbm, dst_hbm, w_hbm, out_hbm,
            src_v, dst_v, w_v, gb0, gb1, gb2, sb0, sb1,
            g0, g1, g2, u0, u1, acc_sh):
    c = lax.axis_index("c")
    s = lax.axis_index("s")
    gbufs = (gb0, gb1, gb2)
    sbufs = (sb0, sb1)
    gsems = (g0, g1, g2)
    usems = (u0, u1)
    table = xs_hbm.at[c]

    def _zero_row(r, _):
        for f in range(DH // 16):
            sb0[r, pl.ds(f * 16, 16)] = jnp.zeros((16,), jnp.float32)
        return 0
    lax.fori_loop(0, ZR, _zero_row, 0)
    zsrc = sb0.at[pl.ds(0, ZR)]
    for k in range(RPT // ZR):
        pltpu.sync_copy(zsrc, acc_sh.at[pl.ds(s * RPT + k * ZR, ZR)])
    plsc.subcore_barrier()

    pltpu.sync_copy(src_hbm.at[s], src_v)
    pltpu.sync_copy(dst_hbm.at[s], dst_v)
    pltpu.sync_copy(w_hbm.at[s], w_v)

    def _issue_gather(t, j):
        pltpu.async_copy(table.at[src_v.at[t]], gbufs[j], gsems[j])

    def _wait_gather(t, j):
        pltpu.make_async_copy(table.at[src_v.at[t]], gbufs[j], gsems[j]).wait()

    def _issue_scatter(t, j):
        pltpu.async_copy(sbufs[j], acc_sh.at[dst_v.at[t]], usems[j], add=True)

    def _wait_scatter(t, j):
        pltpu.make_async_copy(
            sbufs[j], acc_sh.at[dst_v.at[t]], usems[j]).wait()

    ev = 2 * lax.iota(jnp.int32, 16)
    cols = [(ev + g * 32, ev + g * 32 + 1) for g in range(DH // 32)]

    def _scale(t, jg, js):
        gbuf = gbufs[jg]
        sbuf = sbufs[js]

        @plsc.parallel_loop(0, CE, 2, unroll=2)
        def _scale2(e0):
            for jj in range(2):
                e = e0 + jj
                splat_e = jnp.full((16,), e, jnp.int32)
                w_splat = plsc.load_gather(w_v.at[t], [splat_e])
                for g in range(DH // 32):
                    raw = gbuf[e, pl.ds(g * 32, 32)]
                    a, b = plsc.unpack(
                        raw, format=plsc.PackFormat.INTERLEAVED,
                        preferred_element_type=jnp.float32)
                    plsc.store_scatter(sbuf, [splat_e, cols[g][0]],
                                       a * w_splat)
                    plsc.store_scatter(sbuf, [splat_e, cols[g][1]],
                                       b * w_splat)

    def _chunk(t, jg, js, first):
        _wait_gather(t, jg)
        if not first:
            _wait_scatter(t - 2, js)
        _scale(t, jg, js)
        _issue_scatter(t, js)
        if t + 3 < CPT:
            _issue_gather(t + 3, jg)

    _issue_gather(0, 0)
    _issue_gather(1, 1)
    _issue_gather(2, 2)

    for t0 in range(6):
        _chunk(t0, t0 % 3, t0 % 2, t0 < 2)

    def _body(v, _):
        t = v * 6
        for q in range(6):
            _wait_gather(t + q, q % 3)
            _wait_scatter(t + q - 2, q % 2)
            _scale(t + q, q % 3, q % 2)
            _issue_scatter(t + q, q % 2)

            @pl.when(t + q + 3 < CPT)
            def _():
                _issue_gather(t + q + 3, q % 3)
        return 0
    lax.fori_loop(1, CPT // 6, _body, 0)

    for t0 in range(6 * (CPT // 6), CPT):
        _chunk(t0, t0 % 3, t0 % 2, False)
    _wait_scatter(CPT - 2, (CPT - 2) % 2)
    _wait_scatter(CPT - 1, (CPT - 1) % 2)

    plsc.subcore_barrier()
    for k in range(RPT // RCH):
        base = s * RPT + k * RCH
        pltpu.sync_copy(acc_sh.at[pl.ds(base, RCH)],
                        out_hbm.at[c].at[pl.ds(base, RCH)])


_RB = 1000


def _split_perm_bf16(h0, h1, w_ref, xsn_ref):
    xsn_ref[0] = (h0 * w_ref[:, :DH]).astype(jnp.bfloat16)
    xsn_ref[1] = (h1 * w_ref[:, DH:]).astype(jnp.bfloat16)


def _prescale_body(x_ref, w_ref, xs_ref):
    _split_perm_bf16(x_ref[:, :DH], x_ref[:, DH:], w_ref, xs_ref)


def _elem_body(x_ref, agg_ref, deg_ref, w_ref, xn_ref, xsn_ref):
    d = deg_ref[...]
    scale = (1.0 - ALPHA) * d * d
    h0 = ALPHA * x_ref[:, :DH] + scale * agg_ref[0]
    h1 = ALPHA * x_ref[:, DH:] + scale * agg_ref[1]
    inv = 1.0 / (jnp.sum(h0, axis=1, keepdims=True)
                 + jnp.sum(h1, axis=1, keepdims=True))
    h0 = h0 * inv
    h1 = h1 * inv
    xn_ref[:, :DH] = h0
    xn_ref[:, DH:] = h1
    _split_perm_bf16(h0, h1, w_ref, xsn_ref)


def _prescale(x, W):
    return pl.pallas_call(
        _prescale_body,
        grid=(N_NODES // _RB,),
        in_specs=[
            pl.BlockSpec((_RB, D_FEAT), lambda i: (i, 0)),
            pl.BlockSpec((1, D_FEAT), lambda i: (0, 0)),
        ],
        out_specs=pl.BlockSpec((NC, _RB, DH), lambda i: (0, i, 0)),
        out_shape=jax.ShapeDtypeStruct((NC, N_NODES, DH), jnp.bfloat16),
    )(x, W)


def _elemwise(x, agg, deg_col, W):
    return pl.pallas_call(
        _elem_body,
        grid=(N_NODES // _RB,),
        in_specs=[
            pl.BlockSpec((_RB, D_FEAT), lambda i: (i, 0)),
            pl.BlockSpec((NC, _RB, DH), lambda i: (0, i, 0)),
            pl.BlockSpec((_RB, 1), lambda i: (i, 0)),
            pl.BlockSpec((1, D_FEAT), lambda i: (0, 0)),
        ],
        out_specs=[
            pl.BlockSpec((_RB, D_FEAT), lambda i: (i, 0)),
            pl.BlockSpec((NC, _RB, DH), lambda i: (0, i, 0)),
        ],
        out_shape=[
            jax.ShapeDtypeStruct((N_NODES, D_FEAT), jnp.float32),
            jax.ShapeDtypeStruct((NC, N_NODES, DH), jnp.bfloat16),
        ],
    )(x, agg, deg_col, W)


@jax.jit
def kernel(x, edge_index, edge_weight, degree, W):
    src = edge_index[0].astype(jnp.int32)
    dst = edge_index[1].astype(jnp.int32)
    pad = E_PAD - N_EDGES
    src3 = jnp.pad(src, (0, pad)).reshape(NS, CPT, CE)
    dst3 = jnp.pad(dst, (0, pad)).reshape(NS, CPT, CE)
    w3 = jnp.pad(edge_weight, (0, pad)).reshape(NS, CPT, CE)
    deg_col = degree[:, None]

    xs = _prescale(x, W)
    for _ in range(LAYERS):
        agg = _sc_agg(xs, src3, dst3, w3)
        x, xs = _elemwise(x, agg, deg_col, W)
    return x


# speedup vs baseline: 2.6055x; 1.0254x over previous
import functools

import jax
import jax.numpy as jnp
from jax import lax
from jax.experimental import pallas as pl
from jax.experimental.pallas import tpu as pltpu
from jax.experimental.pallas import tpu_sc as plsc

N_NODES = 10000
N_EDGES = 320000
D_FEAT = 128
LAYERS = 3
ALPHA = 0.5

NC = 2
NS = 16
DH = D_FEAT // NC

CE = 128
CPT = 158
EPT = CE * CPT
E_PAD = EPT * NS
NGB = 3
NSB = 2

N_PAD = 10240
RPT = N_PAD // NS
RCH = 128
ZR = 64

_sc_mesh = plsc.VectorSubcoreMesh(
    core_axis_name="c", subcore_axis_name="s", num_cores=NC, num_subcores=NS)


@functools.partial(
    pl.kernel,
    out_type=jax.ShapeDtypeStruct((NC, N_PAD, DH), jnp.float32),
    mesh=_sc_mesh,
    scratch_types=[
        pltpu.VMEM((CPT, CE), jnp.int32),
        pltpu.VMEM((CPT, CE), jnp.int32),
        pltpu.VMEM((CPT, CE), jnp.float32),
        pltpu.VMEM((CE, DH), jnp.bfloat16),
        pltpu.VMEM((CE, DH), jnp.bfloat16),
        pltpu.VMEM((CE, DH), jnp.bfloat16),
        pltpu.VMEM((CE, DH), jnp.float32),
        pltpu.VMEM((CE, DH), jnp.float32),
        pltpu.SemaphoreType.DMA,
        pltpu.SemaphoreType.DMA,
        pltpu.SemaphoreType.DMA,
        pltpu.SemaphoreType.DMA,
        pltpu.SemaphoreType.DMA,
        pltpu.VMEM_SHARED((N_PAD, DH), jnp.float32),
    ],
    compiler_params=pltpu.CompilerParams(
        use_tc_tiling_on_sc=False, needs_layout_passes=False),
)
def _sc_agg(xs_hbm, src_hbm, dst_hbm, w_hbm, out_hbm,
            src_v, dst_v, w_v, gb0, gb1, gb2, sb0, sb1,
            g0, g1, g2, u0, u1, acc_sh):
    c = lax.axis_index("c")
    s = lax.axis_index("s")
    gbufs = (gb0, gb1, gb2)
    sbufs = (sb0, sb1)
    gsems = (g0, g1, g2)
    usems = (u0, u1)
    table = xs_hbm.at[c]

    def _zero_row(r, _):
        for f in range(DH // 16):
            sb0[r, pl.ds(f * 16, 16)] = jnp.zeros((16,), jnp.float32)
        return 0
    lax.fori_loop(0, ZR, _zero_row, 0)
    zsrc = sb0.at[pl.ds(0, ZR)]
    for k in range(RPT // ZR):
        pltpu.async_copy(zsrc, acc_sh.at[pl.ds(s * RPT + k * ZR, ZR)], u0)
    pltpu.async_copy(src_hbm.at[s], src_v, g0)
    pltpu.async_copy(dst_hbm.at[s], dst_v, g1)
    pltpu.async_copy(w_hbm.at[s], w_v, g2)
    for k in range(RPT // ZR):
        pltpu.make_async_copy(
            zsrc, acc_sh.at[pl.ds(s * RPT + k * ZR, ZR)], u0).wait()
    pltpu.make_async_copy(src_hbm.at[s], src_v, g0).wait()
    pltpu.make_async_copy(dst_hbm.at[s], dst_v, g1).wait()
    pltpu.make_async_copy(w_hbm.at[s], w_v, g2).wait()
    plsc.subcore_barrier()

    def _issue_gather(t, j):
        pltpu.async_copy(table.at[src_v.at[t]], gbufs[j], gsems[j])

    def _wait_gather(t, j):
        pltpu.make_async_copy(table.at[src_v.at[t]], gbufs[j], gsems[j]).wait()

    def _issue_scatter(t, j):
        pltpu.async_copy(sbufs[j], acc_sh.at[dst_v.at[t]], usems[j], add=True)

    def _wait_scatter(t, j):
        pltpu.make_async_copy(
            sbufs[j], acc_sh.at[dst_v.at[t]], usems[j]).wait()

    ev = 2 * lax.iota(jnp.int32, 16)
    cols = [(ev + g * 32, ev + g * 32 + 1) for g in range(DH // 32)]

    def _scale(t, jg, js):
        gbuf = gbufs[jg]
        sbuf = sbufs[js]

        @plsc.parallel_loop(0, CE, 2, unroll=4)
        def _scale2(e0):
            for jj in range(2):
                e = e0 + jj
                splat_e = jnp.full((16,), e, jnp.int32)
                w_splat = plsc.load_gather(w_v.at[t], [splat_e])
                for g in range(DH // 32):
                    raw = gbuf[e, pl.ds(g * 32, 32)]
                    a, b = plsc.unpack(
                        raw, format=plsc.PackFormat.INTERLEAVED,
                        preferred_element_type=jnp.float32)
                    plsc.store_scatter(sbuf, [splat_e, cols[g][0]],
                                       a * w_splat)
                    plsc.store_scatter(sbuf, [splat_e, cols[g][1]],
                                       b * w_splat)

    def _chunk(t, jg, js, first):
        _wait_gather(t, jg)
        if not first:
            _wait_scatter(t - 2, js)
        _scale(t, jg, js)
        _issue_scatter(t, js)
        if t + 3 < CPT:
            _issue_gather(t + 3, jg)

    _issue_gather(0, 0)
    _issue_gather(1, 1)
    _issue_gather(2, 2)

    for t0 in range(6):
        _chunk(t0, t0 % 3, t0 % 2, t0 < 2)

    def _body(v, _):
        t = v * 6
        for q in range(6):
            _wait_gather(t + q, q % 3)
            _wait_scatter(t + q - 2, q % 2)
            _scale(t + q, q % 3, q % 2)
            _issue_scatter(t + q, q % 2)

            @pl.when(t + q + 3 < CPT)
            def _():
                _issue_gather(t + q + 3, q % 3)
        return 0
    lax.fori_loop(1, CPT // 6, _body, 0)

    for t0 in range(6 * (CPT // 6), CPT):
        _chunk(t0, t0 % 3, t0 % 2, False)
    _wait_scatter(CPT - 2, (CPT - 2) % 2)
    _wait_scatter(CPT - 1, (CPT - 1) % 2)

    plsc.subcore_barrier()
    for k in range(RPT // RCH):
        base = s * RPT + k * RCH
        pltpu.sync_copy(acc_sh.at[pl.ds(base, RCH)],
                        out_hbm.at[c].at[pl.ds(base, RCH)])


_RB = 1000


def _split_perm_bf16(h0, h1, w_ref, xsn_ref):
    xsn_ref[0] = (h0 * w_ref[:, :DH]).astype(jnp.bfloat16)
    xsn_ref[1] = (h1 * w_ref[:, DH:]).astype(jnp.bfloat16)


def _prescale_body(x_ref, w_ref, xs_ref):
    _split_perm_bf16(x_ref[:, :DH], x_ref[:, DH:], w_ref, xs_ref)


def _elem_body(x_ref, agg_ref, deg_ref, w_ref, xn_ref, xsn_ref):
    d = deg_ref[...]
    scale = (1.0 - ALPHA) * d * d
    h0 = ALPHA * x_ref[:, :DH] + scale * agg_ref[0]
    h1 = ALPHA * x_ref[:, DH:] + scale * agg_ref[1]
    inv = 1.0 / (jnp.sum(h0, axis=1, keepdims=True)
                 + jnp.sum(h1, axis=1, keepdims=True))
    h0 = h0 * inv
    h1 = h1 * inv
    xn_ref[:, :DH] = h0
    xn_ref[:, DH:] = h1
    _split_perm_bf16(h0, h1, w_ref, xsn_ref)


def _prescale(x, W):
    return pl.pallas_call(
        _prescale_body,
        grid=(N_NODES // _RB,),
        in_specs=[
            pl.BlockSpec((_RB, D_FEAT), lambda i: (i, 0)),
            pl.BlockSpec((1, D_FEAT), lambda i: (0, 0)),
        ],
        out_specs=pl.BlockSpec((NC, _RB, DH), lambda i: (0, i, 0)),
        out_shape=jax.ShapeDtypeStruct((NC, N_NODES, DH), jnp.bfloat16),
    )(x, W)


def _elemwise(x, agg, deg_col, W):
    return pl.pallas_call(
        _elem_body,
        grid=(N_NODES // _RB,),
        in_specs=[
            pl.BlockSpec((_RB, D_FEAT), lambda i: (i, 0)),
            pl.BlockSpec((NC, _RB, DH), lambda i: (0, i, 0)),
            pl.BlockSpec((_RB, 1), lambda i: (i, 0)),
            pl.BlockSpec((1, D_FEAT), lambda i: (0, 0)),
        ],
        out_specs=[
            pl.BlockSpec((_RB, D_FEAT), lambda i: (i, 0)),
            pl.BlockSpec((NC, _RB, DH), lambda i: (0, i, 0)),
        ],
        out_shape=[
            jax.ShapeDtypeStruct((N_NODES, D_FEAT), jnp.float32),
            jax.ShapeDtypeStruct((NC, N_NODES, DH), jnp.bfloat16),
        ],
    )(x, agg, deg_col, W)


@jax.jit
def kernel(x, edge_index, edge_weight, degree, W):
    src = edge_index[0].astype(jnp.int32)
    dst = edge_index[1].astype(jnp.int32)
    pad = E_PAD - N_EDGES
    src3 = jnp.pad(src, (0, pad)).reshape(NS, CPT, CE)
    dst3 = jnp.pad(dst, (0, pad)).reshape(NS, CPT, CE)
    w3 = jnp.pad(edge_weight, (0, pad)).reshape(NS, CPT, CE)
    deg_col = degree[:, None]

    xs = _prescale(x, W)
    for _ in range(LAYERS):
        agg = _sc_agg(xs, src3, dst3, w3)
        x, xs = _elemwise(x, agg, deg_col, W)
    return x
